# Initial kernel scaffold; baseline (speedup 1.0000x reference)
import jax
import jax.numpy as jnp
from jax.experimental import pallas as pl


def kernel(x, edge_index, batch, embed, gamma0, beta0, lin_w0, lin_b0, W0, a_src0, a_dst0, conv_b0, gamma1, beta1, lin_w1, lin_b1, W1, a_src1, a_dst1, conv_b1, ro_w, ro_b):
    raise NotImplementedError("write your pallas kernel here")


# speedup vs baseline: 1.0032x; 1.0032x over previous
import functools

import jax
import jax.numpy as jnp
from jax.experimental import pallas as pl
from jax.experimental.pallas import tpu as pltpu

N = 10000
E = 320000
H = 128
OUT = 128
NG = 64


def _readout_body(h_ref, batch_ref, ro_w_ref, ro_b_ref, out_ref):
    h = h_ref[...]
    y = jax.lax.dot_general(h, ro_w_ref[...], (((1,), (1,)), ((), ())),
                            preferred_element_type=jnp.float32)
    b = batch_ref[...].astype(jnp.int32)
    seg = jax.lax.broadcasted_iota(jnp.int32, (NG, N), 0)
    oh = jnp.where(seg == b, 1.0, 0.0).astype(jnp.float32)
    out = jax.lax.dot_general(oh, y, (((1,), (0,)), ((), ())),
                              preferred_element_type=jnp.float32)
    out_ref[...] = out + ro_b_ref[...]


def _readout(h, batch, ro_w, ro_b):
    return pl.pallas_call(
        _readout_body,
        out_shape=jax.ShapeDtypeStruct((NG, OUT), jnp.float32),
        in_specs=[
            pl.BlockSpec(memory_space=pltpu.ANY if False else pltpu.VMEM),
            pl.BlockSpec(memory_space=pltpu.VMEM),
            pl.BlockSpec(memory_space=pltpu.VMEM),
            pl.BlockSpec(memory_space=pltpu.VMEM),
        ],
        out_specs=pl.BlockSpec(memory_space=pltpu.VMEM),
    )(h, batch.reshape(1, N), ro_w, ro_b.reshape(1, OUT))


def _gat(h, src, dst, W, a_src, a_dst, b):
    z = h @ W.T
    e = jax.nn.leaky_relu((z @ a_src)[src] + (z @ a_dst)[dst], 0.2)
    m = jax.ops.segment_max(e, dst, num_segments=N)
    m = jnp.where(jnp.isfinite(m), m, 0.0)
    ex = jnp.exp(e - m[dst])
    denom = jax.ops.segment_sum(ex, dst, num_segments=N)
    alpha = ex / (denom[dst] + 1e-16)
    out = jax.ops.segment_sum(alpha[:, None] * z[src], dst, num_segments=N)
    return out + b


def kernel(x, edge_index, batch, embed, gamma0, beta0, lin_w0, lin_b0, W0,
           a_src0, a_dst0, conv_b0, gamma1, beta1, lin_w1, lin_b1, W1,
           a_src1, a_dst1, conv_b1, ro_w, ro_b):
    src = edge_index[0]
    dst = edge_index[1]
    h = embed[x]
    lp = [
        (gamma0, beta0, lin_w0, lin_b0, W0, a_src0, a_dst0, conv_b0),
        (gamma1, beta1, lin_w1, lin_b1, W1, a_src1, a_dst1, conv_b1),
    ]
    for (gamma, beta, lw, lb, W, asrc, adst, cb) in lp:
        mean = jnp.mean(h, axis=0)
        var = jnp.var(h, axis=0)
        h = (h - mean) / jnp.sqrt(var + 1e-5) * gamma + beta
        h = jax.nn.relu(h @ lw.T + lb)
        h = _gat(h, src, dst, W, asrc, adst, cb)
    return _readout(h, batch, ro_w, ro_b)


# speedup vs baseline: 19.7238x; 19.6605x over previous
import functools

import jax
import jax.numpy as jnp
from jax import lax
from jax.experimental import pallas as pl
from jax.experimental.pallas import tpu as pltpu
from jax.experimental.pallas import tpu_sc as plsc

N = 10000
E = 320000
H = 128
OUT = 128
NG = 64
NEMB = 100

NC = 2
NS = 16
NW = NC * NS
NPAD = 10240
ROWS_PER_TILE = NPAD // NS
EDGES_PER_TILE = E // NW
CHUNK = 80
NCHUNKS = EDGES_PER_TILE // CHUNK


def _f32dot(a, b, dims):
    return lax.dot_general(a, b, (dims, ((), ())),
                           preferred_element_type=jnp.float32)


def _layer_tail(h, gamma, beta, lw, lb, W, asrc, adst):
    mean = jnp.mean(h, axis=0, keepdims=True)
    var = jnp.mean((h - mean) ** 2, axis=0, keepdims=True)
    hn = (h - mean) * lax.rsqrt(var + 1e-5) * gamma + beta
    hl = jnp.maximum(_f32dot(hn, lw, ((1,), (1,))) + lb, 0.0)
    z = _f32dot(hl, W, ((1,), (1,)))
    ssrc = jnp.sum(z * asrc, axis=1, keepdims=True)
    sdst = jnp.sum(z * adst, axis=1, keepdims=True)
    b = jnp.max(ssrc) + jnp.max(sdst)
    b = jnp.where(b >= 0.0, b, 0.2 * b)
    ssrc_pad = jnp.concatenate(
        [ssrc, jnp.zeros((NPAD - N, 1), jnp.float32)], axis=0)
    sdst_pad = jnp.concatenate(
        [sdst, jnp.full((NPAD - N, 1), b, jnp.float32)], axis=0)
    return z, ssrc_pad, sdst_pad


def _merge(accs, dens, cb):
    acc = accs[0] + accs[1]
    den = lax.dot_general(dens, jnp.ones((NW, 1), jnp.float32),
                          (((0,), (0,)), ((), ())),
                          preferred_element_type=jnp.float32)
    return acc[:N] / (den[:N] + 1e-16) + cb


def _pre0_body(xb_ref, emb_ref, g_ref, be_ref, lw_ref, lb_ref, w_ref,
               as_ref, ad_ref, z_out, ss_out, sd_out):
    iota = lax.broadcasted_iota(jnp.int32, (N, NEMB), 1)
    oh = jnp.where(iota == xb_ref[...], 1.0, 0.0).astype(jnp.float32)
    h = _f32dot(oh, emb_ref[...], ((1,), (0,)))
    z, ss, sd = _layer_tail(h, g_ref[...], be_ref[...], lw_ref[...],
                            lb_ref[...], w_ref[...], as_ref[...], ad_ref[...])
    z_out[...] = z
    ss_out[...] = ss
    sd_out[...] = sd


def _mid_body(accs_ref, dens_ref, cb_ref, g_ref, be_ref, lw_ref, lb_ref,
              w_ref, as_ref, ad_ref, z_out, ss_out, sd_out):
    h = _merge(accs_ref[...], dens_ref[...], cb_ref[...])
    z, ss, sd = _layer_tail(h, g_ref[...], be_ref[...], lw_ref[...],
                            lb_ref[...], w_ref[...], as_ref[...], ad_ref[...])
    z_out[...] = z
    ss_out[...] = ss
    sd_out[...] = sd


def _post_body(accs_ref, dens_ref, cb_ref, batch_ref, row_ref, rob_ref,
               out_ref):
    h = _merge(accs_ref[...], dens_ref[...], cb_ref[...])
    y = _f32dot(h, row_ref[...], ((1,), (1,)))
    seg = lax.broadcasted_iota(jnp.int32, (NG, N), 0)
    oh = jnp.where(seg == batch_ref[...], 1.0, 0.0).astype(jnp.float32)
    out = _f32dot(oh, y, ((1,), (0,)))
    cnt = jnp.sum(oh, axis=1, keepdims=True)
    out_ref[...] = out + cnt * rob_ref[...]


def _tc_call(body, out_shapes, *args):
    flat_shapes = jax.tree.leaves(out_shapes)
    return pl.pallas_call(
        body,
        out_shape=out_shapes,
        in_specs=[pl.BlockSpec(memory_space=pltpu.VMEM) for _ in args],
        out_specs=jax.tree.map(
            lambda _: pl.BlockSpec(memory_space=pltpu.VMEM), out_shapes),
    )(*args)


_LAYER_OUT = (jax.ShapeDtypeStruct((N, H), jnp.float32),
              jax.ShapeDtypeStruct((NPAD, 1), jnp.float32),
              jax.ShapeDtypeStruct((NPAD, 1), jnp.float32))


def _sc_edge_body(z_hbm, ssrc_hbm, sdst_hbm, src_hbm, dst_hbm,
                  zero128_hbm, acc_out, den_out,
                  ssrc_v, sdst_v, src_idx, dst_idx, rows, ex_v, den_v,
                  acc_sh, gsem):
    c = lax.axis_index("c")
    s = lax.axis_index("s")
    wid = s * NC + c
    pltpu.sync_copy(ssrc_hbm, ssrc_v)
    pltpu.sync_copy(sdst_hbm, sdst_v)
    rbase = s * ROWS_PER_TILE
    pltpu.sync_copy(zero128_hbm, acc_sh.at[pl.ds(rbase, ROWS_PER_TILE)])

    def zero_body(i, carry):
        den_v[pl.ds(i * 16, 16)] = jnp.zeros((16,), jnp.float32)
        return carry

    lax.fori_loop(0, NPAD // 16, zero_body, 0, unroll=False)
    plsc.subcore_barrier()
    bvec = sdst_v[pl.ds(NPAD - 16, 16)]
    ebase = wid * EDGES_PER_TILE

    def chunk_body(i, carry):
        off = ebase + i * CHUNK
        pltpu.sync_copy(src_hbm.at[pl.ds(off, CHUNK)], src_idx)
        pltpu.sync_copy(dst_hbm.at[pl.ds(off, CHUNK)], dst_idx)
        gather = pltpu.async_copy(z_hbm.at[src_idx], rows, gsem)
        for j in range(CHUNK // 16):
            sv = src_idx[pl.ds(j * 16, 16)]
            dv = dst_idx[pl.ds(j * 16, 16)]
            e = plsc.load_gather(ssrc_v, [sv]) + plsc.load_gather(sdst_v, [dv])
            e = jnp.where(e >= 0.0, e, 0.2 * e)
            ex = jnp.exp(e - bvec)
            ex_v[pl.ds(j * 16, 16)] = ex
            plsc.addupdate_scatter(den_v, [dv], ex)
        gather.wait()

        def scale_body(k, carry2):
            exb = plsc.load_gather(ex_v, [jnp.full((16,), k, jnp.int32)])
            for j in range(H // 16):
                sl = pl.ds(j * 16, 16)
                rows[k, sl] = rows[k, sl] * exb
            return carry2

        lax.fori_loop(0, CHUNK, scale_body, 0, unroll=False)
        pltpu.sync_copy(rows, acc_sh.at[dst_idx], add=True)
        return carry

    lax.fori_loop(0, NCHUNKS, chunk_body, 0, unroll=False)
    plsc.subcore_barrier()
    pltpu.sync_copy(acc_sh.at[pl.ds(rbase, ROWS_PER_TILE)],
                    acc_out.at[c].at[pl.ds(rbase, ROWS_PER_TILE)])
    pltpu.sync_copy(den_v, den_out.at[c].at[s])


_sc_edge = pl.kernel(
    _sc_edge_body,
    out_type=(jax.ShapeDtypeStruct((NC, NPAD, H), jnp.float32),
              jax.ShapeDtypeStruct((NC, NS, NPAD), jnp.float32)),
    mesh=plsc.VectorSubcoreMesh(core_axis_name="c", subcore_axis_name="s"),
    compiler_params=pltpu.CompilerParams(needs_layout_passes=False),
    scratch_types=[
        pltpu.VMEM((NPAD,), jnp.float32),
        pltpu.VMEM((NPAD,), jnp.float32),
        pltpu.VMEM((CHUNK,), jnp.int32),
        pltpu.VMEM((CHUNK,), jnp.int32),
        pltpu.VMEM((CHUNK, H), jnp.float32),
        pltpu.VMEM((CHUNK,), jnp.float32),
        pltpu.VMEM((NPAD,), jnp.float32),
        pltpu.VMEM_SHARED((NPAD, H), jnp.float32),
        pltpu.SemaphoreType.DMA,
    ],
)


def kernel(x, edge_index, batch, embed, gamma0, beta0, lin_w0, lin_b0, W0,
           a_src0, a_dst0, conv_b0, gamma1, beta1, lin_w1, lin_b1, W1,
           a_src1, a_dst1, conv_b1, ro_w, ro_b):
    src = edge_index[0].astype(jnp.int32)
    dst = edge_index[1].astype(jnp.int32)
    xb = x.reshape(N, 1).astype(jnp.int32)
    zero128 = jnp.zeros((ROWS_PER_TILE, H), jnp.float32)

    z0, ss0, sd0 = _tc_call(
        _pre0_body, _LAYER_OUT, xb, embed, gamma0.reshape(1, H),
        beta0.reshape(1, H), lin_w0, lin_b0.reshape(1, H), W0,
        a_src0.reshape(1, H), a_dst0.reshape(1, H))
    accs0, dens0 = _sc_edge(z0, ss0.reshape(NPAD), sd0.reshape(NPAD),
                            src, dst, zero128)
    z1, ss1, sd1 = _tc_call(
        _mid_body, _LAYER_OUT, accs0, dens0.reshape(NW, NPAD),
        conv_b0.reshape(1, H), gamma1.reshape(1, H), beta1.reshape(1, H),
        lin_w1, lin_b1.reshape(1, H), W1, a_src1.reshape(1, H),
        a_dst1.reshape(1, H))
    accs1, dens1 = _sc_edge(z1, ss1.reshape(NPAD), sd1.reshape(NPAD),
                            src, dst, zero128)
    out = _tc_call(
        _post_body, jax.ShapeDtypeStruct((NG, OUT), jnp.float32),
        accs1, dens1.reshape(NW, NPAD), conv_b1.reshape(1, H),
        batch.reshape(1, N).astype(jnp.int32), ro_w, ro_b.reshape(1, OUT))
    return out
